# Initial kernel scaffold; baseline (speedup 1.0000x reference)
import jax
import jax.numpy as jnp
from jax.experimental import pallas as pl


def kernel(x, edge_index, batch, c1w, c1b, bn1g, bn1b, c2w, c2b, bn2g, bn2b, c3w, c3b, bn3g, bn3b, g1w, g1b, g2w, g2b, clf1w, clf1b, clf2w, clf2b, mag1w, mag1b, mag2w, mag2b):
    raise NotImplementedError("write your pallas kernel here")


# speedup vs baseline: 4.5810x; 4.5810x over previous
import functools

import jax
import jax.numpy as jnp
from jax import lax
from jax.experimental import pallas as pl
from jax.experimental.pallas import tpu as pltpu
from jax.experimental.pallas import tpu_sc as plsc

F32 = jnp.float32

N_NODES = 10000
HID = 64
NB = 20
BN = N_NODES // NB
J = 32


def _build_w1(w1):
    W = jnp.zeros((48, 128), F32)
    for k in range(7):
        for r1 in range(8):
            t = 2 * r1 + k - 3
            a, r0 = t // 16, t % 16
            row = (a + 1) * 16 + r0
            W = W.at[row, r1 * 16:(r1 + 1) * 16].add(w1[:, 0, k])
    return W


def _build_w2(w2):
    W = jnp.zeros((176, 128), F32)
    for k in range(5):
        for r2 in range(4):
            t = 2 * r2 + k - 2
            ti = t + 2
            W = W.at[ti * 16:(ti + 1) * 16, r2 * 32:(r2 + 1) * 32].add(w2[:, :, k].T)
    return W


def _build_w3(w3):
    W = jnp.zeros((160, 128), F32)
    for k in range(3):
        for r3 in range(2):
            t = 2 * r3 + k - 1
            ti = t + 1
            W = W.at[ti * 32:(ti + 1) * 32, r3 * 64:(r3 + 1) * 64].add(w3[:, :, k].T)
    return W


def _accum_stats(y, s_ref):
    @pl.when(pl.program_id(0) == 0)
    def _():
        s_ref[...] = jnp.zeros_like(s_ref)
    s = jnp.sum(y, axis=(0, 1)).reshape(1, 128)
    q = jnp.sum(y * y, axis=(0, 1)).reshape(1, 128)
    s_ref[...] += jnp.concatenate([s, q], axis=0)


def _k1_body(x_ref, w_ref, y_ref, s_ref):
    xb = x_ref[...]
    xp = jnp.pad(xb, ((0, 0), (1, 1), (0, 0)))
    him = jnp.concatenate([xp[:, a:a + J, :] for a in range(3)], axis=-1)
    y = lax.dot_general(him.reshape(BN * J, 48), w_ref[...],
                        (((1,), (0,)), ((), ())),
                        preferred_element_type=F32).reshape(BN, J, 128)
    y_ref[...] = y
    _accum_stats(y, s_ref)


def _k23_body(slices, kdim, y_ref, sc_ref, sh_ref, w_ref, o_ref, s_ref):
    h = jnp.maximum(y_ref[...] * sc_ref[...] + sh_ref[...], 0.0)
    hp = jnp.pad(h, ((0, 0), (1, 1), (0, 0)))
    him = jnp.concatenate(
        [hp[:, 1 + sj:1 + sj + J, c0:c0 + cw] for (sj, c0, cw) in slices],
        axis=-1)
    y = lax.dot_general(him.reshape(BN * J, kdim), w_ref[...],
                        (((1,), (0,)), ((), ())),
                        preferred_element_type=F32).reshape(BN, J, 128)
    o_ref[...] = y
    _accum_stats(y, s_ref)


def _k4_body(y_ref, sc_ref, sh_ref, f_ref):
    h = jnp.maximum(y_ref[...] * sc_ref[...] + sh_ref[...], 0.0)
    s = jnp.sum(h, axis=1)
    f_ref[0] = (s[:, 0:64] + s[:, 64:128]) * (1.0 / 64.0)


def _bn_scale_shift(stats, g, b, phases, cnt):
    c = g.shape[0]
    s = stats[0].reshape(phases, c).sum(axis=0)
    q = stats[1].reshape(phases, c).sum(axis=0)
    m = s / cnt
    v = q / cnt - m * m
    inv = g * lax.rsqrt(v + 1e-5)
    sc = jnp.tile(inv, phases).reshape(1, 128)
    sh = jnp.tile(b - m * inv, phases).reshape(1, 128)
    return sc, sh


def _cnn_feat(x, c1w, c2w, c3w, bn1g, bn1b, bn2g, bn2b, bn3g, bn3b):
    x16 = x.reshape(N_NODES, J, 16)
    w1e, w2e, w3e = _build_w1(c1w), _build_w2(c2w), _build_w3(c3w)

    blk = lambda shp: pl.BlockSpec(shp, lambda i: (i, 0, 0))
    cst = lambda shp: pl.BlockSpec(shp, lambda i: tuple(0 for _ in shp))

    y1, st1 = pl.pallas_call(
        _k1_body,
        grid=(NB,),
        in_specs=[blk((BN, J, 16)), cst((48, 128))],
        out_specs=[blk((BN, J, 128)), cst((2, 128))],
        out_shape=[jax.ShapeDtypeStruct((N_NODES, J, 128), F32),
                   jax.ShapeDtypeStruct((2, 128), F32)],
    )(x16, w1e)
    sc1, sh1 = _bn_scale_shift(st1, bn1g, bn1b, 8, N_NODES * 256.0)

    sl2 = tuple((t // 8, (t % 8) * 16, 16) for t in range(-2, 9))
    y2, st2 = pl.pallas_call(
        functools.partial(_k23_body, sl2, 176),
        grid=(NB,),
        in_specs=[blk((BN, J, 128)), cst((1, 128)), cst((1, 128)), cst((176, 128))],
        out_specs=[blk((BN, J, 128)), cst((2, 128))],
        out_shape=[jax.ShapeDtypeStruct((N_NODES, J, 128), F32),
                   jax.ShapeDtypeStruct((2, 128), F32)],
    )(y1, sc1, sh1, w2e)
    sc2, sh2 = _bn_scale_shift(st2, bn2g, bn2b, 4, N_NODES * 128.0)

    sl3 = tuple((t // 4, (t % 4) * 32, 32) for t in range(-1, 4))
    y3, st3 = pl.pallas_call(
        functools.partial(_k23_body, sl3, 160),
        grid=(NB,),
        in_specs=[blk((BN, J, 128)), cst((1, 128)), cst((1, 128)), cst((160, 128))],
        out_specs=[blk((BN, J, 128)), cst((2, 128))],
        out_shape=[jax.ShapeDtypeStruct((N_NODES, J, 128), F32),
                   jax.ShapeDtypeStruct((2, 128), F32)],
    )(y2, sc2, sh2, w3e)
    sc3, sh3 = _bn_scale_shift(st3, bn3g, bn3b, 2, N_NODES * 64.0)

    feat = pl.pallas_call(
        _k4_body,
        grid=(NB,),
        in_specs=[blk((BN, J, 128)), cst((1, 128)), cst((1, 128))],
        out_specs=pl.BlockSpec((1, BN, 64), lambda i: (i, 0, 0)),
        out_shape=jax.ShapeDtypeStruct((NB, BN, 64), F32),
    )(y3, sc3, sh3)
    return feat.reshape(N_NODES, 64)


_CH = 80
NPAD = 10240
NPC = NPAD // 2
NROW = 2688
RPT = NROW // 16
TRASH = 2560


def _remap(dstv, idxv, nbase, parity):
    for j in range(_CH // 16):
        d16 = dstv[pl.ds(j * 16, 16)] - nbase
        inb = jnp.logical_and(d16 >= 0, d16 < NPC)
        sel = jnp.logical_and(inb, jnp.bitwise_and(d16, 1) == parity)
        h = lax.shift_right_logical(d16, 1)
        idxv[pl.ds(j * 16, 16)] = jnp.where(sel, h, TRASH)


def _sc_degree(dst, e):
    ept = e // 16
    nch = ept // _CH
    mesh = plsc.VectorSubcoreMesh(core_axis_name="c", subcore_axis_name="s")

    @functools.partial(
        pl.kernel, mesh=mesh,
        out_type=jax.ShapeDtypeStruct((2, NROW, 128), F32),
        scratch_types=[
            pltpu.VMEM((_CH,), jnp.int32),
            pltpu.VMEM((_CH,), jnp.int32),
            pltpu.VMEM((_CH, 128), F32),
            pltpu.VMEM((_CH, 128), F32),
            pltpu.VMEM_SHARED((NROW, 128), F32),
        ],
    )
    def deg_kernel(dst_hbm, onl_hbm, onh_hbm, z_hbm, out_hbm,
                   dstv, idxv, onl, onh, acc):
        cid = lax.axis_index("c")
        sid = lax.axis_index("s")
        roff = sid * RPT
        pltpu.sync_copy(onl_hbm, onl)
        pltpu.sync_copy(onh_hbm, onh)
        pltpu.sync_copy(z_hbm, acc.at[pl.ds(roff, RPT)])
        plsc.subcore_barrier()
        nbase = cid * NPC
        base0 = sid * ept

        def chunk(i, carry):
            b = base0 + i * _CH
            pltpu.sync_copy(dst_hbm.at[pl.ds(b, _CH)], dstv)
            _remap(dstv, idxv, nbase, 0)
            pltpu.sync_copy(onl, acc.at[idxv], add=True)
            _remap(dstv, idxv, nbase, 1)
            pltpu.sync_copy(onh, acc.at[idxv], add=True)
            return carry

        lax.fori_loop(0, nch, chunk, 0)
        plsc.subcore_barrier()
        pltpu.sync_copy(acc.at[pl.ds(roff, RPT)], out_hbm.at[cid, pl.ds(roff, RPT)])

    col = jnp.arange(128)
    onl = jnp.where(col < 64, 1.0, 0.0).astype(F32) * jnp.ones((_CH, 1), F32)
    onh = jnp.where(col >= 64, 1.0, 0.0).astype(F32) * jnp.ones((_CH, 1), F32)
    zeros = jnp.zeros((RPT, 128), F32)
    return deg_kernel(dst, onl, onh, zeros)


def _sc_scatter(m, src, dst, e):
    ept = e // 16
    nch = ept // _CH
    mesh = plsc.VectorSubcoreMesh(core_axis_name="c", subcore_axis_name="s")

    @functools.partial(
        pl.kernel, mesh=mesh,
        out_type=jax.ShapeDtypeStruct((2, NROW, 128), F32),
        scratch_types=[
            pltpu.VMEM((_CH,), jnp.int32),
            pltpu.VMEM((_CH,), jnp.int32),
            pltpu.VMEM((_CH,), jnp.int32),
            pltpu.VMEM((_CH, 128), F32),
            pltpu.VMEM((_CH, 128), F32),
            pltpu.VMEM_SHARED((NROW, 128), F32),
            pltpu.SemaphoreType.DMA,
        ],
    )
    def msg_kernel(m_hbm, src_hbm, dst_hbm, z_hbm, out_hbm,
                   srcv, dstv, idxv, rows, rows2, acc, sem):
        cid = lax.axis_index("c")
        sid = lax.axis_index("s")
        roff = sid * RPT
        pltpu.sync_copy(z_hbm, acc.at[pl.ds(roff, RPT)])

        def zlow(r, carry):
            for q in range(4):
                rows2[r, pl.ds(q * 16, 16)] = jnp.zeros((16,), F32)
            return carry

        lax.fori_loop(0, _CH, zlow, 0)
        plsc.subcore_barrier()
        nbase = cid * NPC
        base0 = sid * ept

        def chunk(i, carry):
            b = base0 + i * _CH
            pltpu.sync_copy(src_hbm.at[pl.ds(b, _CH)], srcv)
            pltpu.sync_copy(dst_hbm.at[pl.ds(b, _CH)], dstv)
            pltpu.async_copy(m_hbm.at[srcv], rows, sem).wait()

            def shift(r, c2):
                for q in range(4):
                    rows2[r, pl.ds(64 + q * 16, 16)] = rows[r, pl.ds(q * 16, 16)]
                return c2

            lax.fori_loop(0, _CH, shift, 0)
            _remap(dstv, idxv, nbase, 0)
            pltpu.sync_copy(rows, acc.at[idxv], add=True)
            _remap(dstv, idxv, nbase, 1)
            pltpu.sync_copy(rows2, acc.at[idxv], add=True)
            return carry

        lax.fori_loop(0, nch, chunk, 0)
        plsc.subcore_barrier()
        pltpu.sync_copy(acc.at[pl.ds(roff, RPT)], out_hbm.at[cid, pl.ds(roff, RPT)])

    zeros = jnp.zeros((RPT, 128), F32)
    return msg_kernel(m, src, dst, zeros)


def _unpack_acc(accp_ref):
    parts = []
    for c in range(2):
        sub = accp_ref[c][:NPC // 2]
        ev = sub[:, 0:64]
        od = sub[:, 64:128]
        parts.append(jnp.stack([ev, od], axis=1).reshape(NPC, 64))
    return jnp.concatenate(parts, axis=0)[:N_NODES]


def _g1_body(feat_ref, degp_ref, w_ref, dinv_ref, m_ref):
    deg = _unpack_acc(degp_ref)
    dinv = lax.rsqrt(1.0 + deg[:, 0:1])
    dinv_ref[...] = dinv
    m = lax.dot_general(feat_ref[...], w_ref[...],
                        (((1,), (0,)), ((), ())),
                        preferred_element_type=F32) * dinv
    m_ref[...] = jnp.pad(m, ((0, 0), (0, 64)))


def _g2_body(accp_ref, mprev_ref, dinv_ref, b_ref, w_ref, m_ref):
    dinv = dinv_ref[...]
    acc = _unpack_acc(accp_ref)
    mprev = mprev_ref[...][:, :HID]
    xnew = jnp.maximum((acc + mprev) * dinv + b_ref[...], 0.0)
    m = lax.dot_general(xnew, w_ref[...],
                        (((1,), (0,)), ((), ())),
                        preferred_element_type=F32) * dinv
    m_ref[...] = jnp.pad(m, ((0, 0), (0, 64)))


def _g3_body(accp_ref, mprev_ref, dinv_ref, b_ref, batch_ref,
             c1w_ref, c1b_ref, c2w_ref, c2b_ref,
             m1w_ref, m1b_ref, m2w_ref, m2b_ref,
             logits_ref, mag_ref):
    acc = _unpack_acc(accp_ref)
    mprev = mprev_ref[...][:, :HID]
    h2 = jnp.maximum(
        (acc + mprev) * dinv_ref[...] + b_ref[...], 0.0)
    gids = lax.broadcasted_iota(jnp.int32, (1, 32), 1)
    oh = (batch_ref[...] == gids).astype(F32)
    sums = lax.dot_general(oh, h2, (((0,), (0,)), ((), ())),
                           preferred_element_type=F32)
    cnt = jnp.sum(oh, axis=0)
    ge = sums / jnp.clip(cnt, 1.0)[:, None]

    def head(w1, b1, w2, b2):
        t = jnp.maximum(lax.dot_general(ge, w1, (((1,), (0,)), ((), ())),
                                        preferred_element_type=F32) + b1, 0.0)
        return lax.dot_general(t, w2, (((1,), (0,)), ((), ())),
                               preferred_element_type=F32) + b2

    logits_ref[...] = head(c1w_ref[...], c1b_ref[...], c2w_ref[...], c2b_ref[...])
    mag_ref[...] = head(m1w_ref[...], m1b_ref[...], m2w_ref[...], m2b_ref[...])


def kernel(x, edge_index, batch, c1w, c1b, bn1g, bn1b, c2w, c2b, bn2g, bn2b,
           c3w, c3b, bn3g, bn3b, g1w, g1b, g2w, g2b, clf1w, clf1b, clf2w,
           clf2b, mag1w, mag1b, mag2w, mag2b):
    e = edge_index.shape[1]
    src = edge_index[0]
    dst = edge_index[1]

    feat = _cnn_feat(x, c1w, c2w, c3w, bn1g, bn1b, bn2g, bn2b, bn3g, bn3b)

    degp = _sc_degree(dst, e)

    dinv, m1 = pl.pallas_call(
        _g1_body,
        out_shape=[jax.ShapeDtypeStruct((N_NODES, 1), F32),
                   jax.ShapeDtypeStruct((N_NODES, 128), F32)],
    )(feat, degp, g1w)

    acc1 = _sc_scatter(m1, src, dst, e)

    m2 = pl.pallas_call(
        _g2_body,
        out_shape=jax.ShapeDtypeStruct((N_NODES, 128), F32),
    )(acc1, m1, dinv, g1b, g2w)

    acc2 = _sc_scatter(m2, src, dst, e)

    logits, mag = pl.pallas_call(
        _g3_body,
        out_shape=[jax.ShapeDtypeStruct((32, 2), F32),
                   jax.ShapeDtypeStruct((32, 1), F32)],
    )(acc2, m2, dinv, g2b, batch.reshape(N_NODES, 1),
      clf1w, clf1b, clf2w, clf2b, mag1w, mag1b, mag2w, mag2b)
    return (logits, mag)
